# baseline (speedup 1.0000x reference)
import jax
import jax.numpy as jnp
from jax.experimental import pallas as pl
from jax.experimental.pallas import tpu as pltpu

_B = 16


def _copy_add_block(upd_ref, x_ref, o_ref):
    o_ref[...] = x_ref[...]
    o_ref[0, :, 18:20] = o_ref[0, :, 18:20] + upd_ref[0, 1, :]
    o_ref[0, :, 34:36] = o_ref[0, :, 34:36] + upd_ref[0, 0, :]
    o_ref[0, :, 50:52] = o_ref[0, :, 50:52] + upd_ref[0, 2, :]


def kernel(x, update):
    xv = x.reshape(4, 256, 2048 * 16)
    upd = update.reshape(4, 3, 2)
    out = pl.pallas_call(
        _copy_add_block,
        grid=(4, 256 // _B),
        in_specs=[
            pl.BlockSpec((1, 3, 2), lambda b, j: (b, 0, 0)),
            pl.BlockSpec((1, _B, 2048 * 16), lambda b, j: (b, j, 0)),
        ],
        out_specs=pl.BlockSpec((1, _B, 2048 * 16), lambda b, j: (b, j, 0)),
        out_shape=jax.ShapeDtypeStruct(xv.shape, xv.dtype),
    )(upd, xv)
    return out.reshape(x.shape)
